# baseline (speedup 1.0000x reference)
import jax
import jax.numpy as jnp
from jax import lax
from jax.experimental import pallas as pl
from jax.experimental.pallas import tpu as pltpu
from jax.experimental.pallas import tpu_sc as plsc

N_ENTITIES = 100000
N_RELATIONS = 237
C = 50
WP = 40
NPAIR = 28
E = 1600000
NW = 32
EPW = E // NW
B = 400
NB = 3 * B
NCHUNK = EPW // B
NGROUP = B // 16
CU = 10


def _dist_mult_body(packed_hbm, table_hbm, rel_hbm, out_hbm,
                    idx_v0, idx_v1, rows_v0, rows_v1, strip_v,
                    out_v0, out_v1,
                    sem_i0, sem_i1, sem_g0, sem_g1, sem_o0, sem_o1):
    idx_v = (idx_v0, idx_v1)
    rows_v = (rows_v0, rows_v1)
    out_v = (out_v0, out_v1)
    sem_i = (sem_i0, sem_i1)
    sem_g = (sem_g0, sem_g1)
    sem_o = (sem_o0, sem_o1)

    nc = 2
    wid = lax.axis_index("s") * nc + lax.axis_index("c")

    iota16 = lax.iota(jnp.int32, 16)

    def fire_idx(i, p):
        pltpu.async_copy(packed_hbm.at[wid * NCHUNK + i], idx_v[p], sem_i[p])

    def wait_idx(p):
        pltpu.make_async_copy(packed_hbm.at[0], idx_v[p], sem_i[p]).wait()

    def fire_gathers(p):
        pltpu.async_copy(table_hbm.at[idx_v[p].at[pl.ds(0, B)]],
                         rows_v[p].at[pl.ds(0, B)], sem_g[p])
        pltpu.async_copy(table_hbm.at[idx_v[p].at[pl.ds(B, B)]],
                         rows_v[p].at[pl.ds(B, B)], sem_g[p])
        pltpu.async_copy(rel_hbm.at[idx_v[p].at[pl.ds(2 * B, B)]],
                         rows_v[p].at[pl.ds(2 * B, B)], sem_g[p])

    def wait_gathers(p):
        for j in range(3):
            pltpu.make_async_copy(table_hbm.at[idx_v[p].at[pl.ds(j * B, B)]],
                                  rows_v[p].at[pl.ds(j * B, B)],
                                  sem_g[p]).wait()

    def fire_out(i, p):
        base = wid * EPW + i * B
        pltpu.async_copy(out_v[p], out_hbm.at[pl.ds(base, B)], sem_o[p])

    def wait_out(p):
        pltpu.make_async_copy(out_v[p], out_hbm.at[pl.ds(0, B)],
                              sem_o[p]).wait()

    def compute(p):
        rows = rows_v[p]
        idx = idx_v[p]
        outb = out_v[p]

        zf = jnp.zeros((16,), jnp.float32)
        le16 = jnp.right_shift(iota16, 2)
        lc16 = jnp.bitwise_and(iota16, 3)

        def group_body(g, carry):
            for j in range(4):
                e4 = g * 16 + j * 4 + le16
                be4 = B + e4
                re4 = 2 * B + e4
                a0 = zf
                a1 = zf
                cvec = lc16
                for k in range(7):
                    s2 = plsc.bitcast(plsc.load_gather(rows, [e4, cvec]),
                                      jnp.bfloat16)
                    d2 = plsc.bitcast(plsc.load_gather(rows, [be4, cvec]),
                                      jnp.bfloat16)
                    r2 = plsc.bitcast(plsc.load_gather(rows, [re4, cvec]),
                                      jnp.bfloat16)
                    sd = s2 * d2
                    pa, pb = plsc.unpack(sd, format=plsc.PackFormat.INTERLEAVED)
                    ra, rb = plsc.unpack(r2, format=plsc.PackFormat.INTERLEAVED)
                    a0 = a0 + pa * ra
                    a1 = a1 + pb * rb
                    cvec = cvec + 4
                strip_v[pl.ds(j * 16, 16)] = a0 + a1
            a4 = iota16 * 4
            sc = zf
            for lc in range(4):
                sc = sc + plsc.load_gather(strip_v, [a4 + lc])
            outb[pl.ds(g * 16, 16)] = sc
            return carry

        lax.fori_loop(0, NGROUP, group_body, 0, unroll=False)

    fire_idx(0, 0)
    wait_idx(0)
    fire_gathers(0)
    fire_idx(1, 1)

    def pair_body(k, carry):
        i0 = 2 * k
        wait_gathers(0)
        wait_idx(1)
        fire_gathers(1)

        @pl.when(k > 0)
        def _():
            wait_out(0)
        compute(0)
        fire_out(i0, 0)
        fire_idx(i0 + 2, 0)

        wait_gathers(1)
        wait_idx(0)
        fire_gathers(0)

        @pl.when(k > 0)
        def _():
            wait_out(1)
        compute(1)
        fire_out(i0 + 1, 1)

        @pl.when(i0 + 3 < NCHUNK)
        def _():
            fire_idx(i0 + 3, 1)
        return carry

    lax.fori_loop(0, (NCHUNK - 1) // 2, pair_body, 0, unroll=False)

    wait_gathers(0)
    wait_out(0)
    compute(0)
    fire_out(NCHUNK - 1, 0)
    wait_out(0)
    wait_out(1)


@jax.jit
def _dist_mult(packed, table, rel):
    mesh = plsc.VectorSubcoreMesh(core_axis_name="c", subcore_axis_name="s")
    return pl.kernel(
        _dist_mult_body,
        out_type=jax.ShapeDtypeStruct((E,), jnp.float32),
        mesh=mesh,
        scratch_types=[
            pltpu.VMEM((NB,), jnp.int32),
            pltpu.VMEM((NB,), jnp.int32),
            pltpu.VMEM((3 * B, WP), jnp.int32),
            pltpu.VMEM((3 * B, WP), jnp.int32),
            pltpu.VMEM((64,), jnp.float32),
            pltpu.VMEM((B,), jnp.float32),
            pltpu.VMEM((B,), jnp.float32),
            pltpu.SemaphoreType.DMA,
            pltpu.SemaphoreType.DMA,
            pltpu.SemaphoreType.DMA,
            pltpu.SemaphoreType.DMA,
            pltpu.SemaphoreType.DMA,
            pltpu.SemaphoreType.DMA,
        ],
        compiler_params=pltpu.CompilerParams(
            needs_layout_passes=False, use_tc_tiling_on_sc=False),
    )(packed, table, rel)


def _pack_bf16(x):
    xb = jnp.pad(x.astype(jnp.bfloat16), ((0, 0), (0, 2 * WP - C)))
    return jax.lax.bitcast_convert_type(
        xb.reshape(x.shape[0], WP, 2), jnp.int32)


def kernel(edge_index, edge_type, initializations, rel_emb):
    table = _pack_bf16(initializations)
    rel = _pack_bf16(rel_emb)
    packed = jnp.concatenate(
        [edge_index[0].reshape(NW * NCHUNK, B),
         edge_index[1].reshape(NW * NCHUNK, B),
         edge_type.reshape(NW * NCHUNK, B)], axis=1)
    return _dist_mult(packed, table, rel)


# speedup vs baseline: 1.0001x; 1.0001x over previous
import jax
import jax.numpy as jnp
from jax import lax
from jax.experimental import pallas as pl
from jax.experimental.pallas import tpu as pltpu
from jax.experimental.pallas import tpu_sc as plsc

N_ENTITIES = 100000
N_RELATIONS = 237
C = 50
WP = 40
NPAIR = 28
E = 1600000
NW = 32
EPW = E // NW
B = 400
NB = 3 * B
NCHUNK = EPW // B
NGROUP = B // 16
CU = 10


def _dist_mult_body(packed_hbm, table_hbm, rel_hbm, out_hbm,
                    idx_v0, idx_v1, rows_v0, rows_v1, strip_v,
                    out_v0, out_v1,
                    sem_i0, sem_i1, sem_g0, sem_g1, sem_o0, sem_o1):
    idx_v = (idx_v0, idx_v1)
    rows_v = (rows_v0, rows_v1)
    out_v = (out_v0, out_v1)
    sem_i = (sem_i0, sem_i1)
    sem_g = (sem_g0, sem_g1)
    sem_o = (sem_o0, sem_o1)

    nc = 2
    wid = lax.axis_index("s") * nc + lax.axis_index("c")

    iota16 = lax.iota(jnp.int32, 16)

    def fire_idx(i, p):
        pltpu.async_copy(packed_hbm.at[wid * NCHUNK + i], idx_v[p], sem_i[p])

    def wait_idx(p):
        pltpu.make_async_copy(packed_hbm.at[0], idx_v[p], sem_i[p]).wait()

    def fire_gathers(p):
        pltpu.async_copy(table_hbm.at[idx_v[p].at[pl.ds(0, B)]],
                         rows_v[p].at[pl.ds(0, B)], sem_g[p])
        pltpu.async_copy(table_hbm.at[idx_v[p].at[pl.ds(B, B)]],
                         rows_v[p].at[pl.ds(B, B)], sem_g[p])
        pltpu.async_copy(rel_hbm.at[idx_v[p].at[pl.ds(2 * B, B)]],
                         rows_v[p].at[pl.ds(2 * B, B)], sem_g[p])

    def wait_gathers(p):
        for j in range(3):
            pltpu.make_async_copy(table_hbm.at[idx_v[p].at[pl.ds(j * B, B)]],
                                  rows_v[p].at[pl.ds(j * B, B)],
                                  sem_g[p]).wait()

    def fire_out(i, p):
        base = wid * EPW + i * B
        pltpu.async_copy(out_v[p], out_hbm.at[pl.ds(base, B)], sem_o[p])

    def wait_out(p):
        pltpu.make_async_copy(out_v[p], out_hbm.at[pl.ds(0, B)],
                              sem_o[p]).wait()

    def compute(p):
        rows = rows_v[p]
        idx = idx_v[p]
        outb = out_v[p]

        zf = jnp.zeros((16,), jnp.float32)
        le16 = jnp.right_shift(iota16, 2)
        lc16 = jnp.bitwise_and(iota16, 3)

        def group_body(g, carry):
            for j in range(4):
                e4 = g * 16 + j * 4 + le16
                be4 = B + e4
                re4 = 2 * B + e4
                a0 = zf
                a1 = zf
                cvec = lc16
                for k in range(7):
                    s2 = plsc.bitcast(plsc.load_gather(rows, [e4, cvec]),
                                      jnp.bfloat16)
                    d2 = plsc.bitcast(plsc.load_gather(rows, [be4, cvec]),
                                      jnp.bfloat16)
                    r2 = plsc.bitcast(plsc.load_gather(rows, [re4, cvec]),
                                      jnp.bfloat16)
                    srd = s2 * d2 * r2
                    pa, pb = plsc.unpack(srd,
                                         format=plsc.PackFormat.INTERLEAVED)
                    a0 = a0 + pa
                    a1 = a1 + pb
                    cvec = cvec + 4
                strip_v[pl.ds(j * 16, 16)] = a0 + a1
            a4 = iota16 * 4
            sc = zf
            for lc in range(4):
                sc = sc + plsc.load_gather(strip_v, [a4 + lc])
            outb[pl.ds(g * 16, 16)] = sc
            return carry

        lax.fori_loop(0, NGROUP, group_body, 0, unroll=False)

    fire_idx(0, 0)
    wait_idx(0)
    fire_gathers(0)
    fire_idx(1, 1)

    def pair_body(k, carry):
        i0 = 2 * k
        wait_gathers(0)
        wait_idx(1)
        fire_gathers(1)

        @pl.when(k > 0)
        def _():
            wait_out(0)
        compute(0)
        fire_out(i0, 0)
        fire_idx(i0 + 2, 0)

        wait_gathers(1)
        wait_idx(0)
        fire_gathers(0)

        @pl.when(k > 0)
        def _():
            wait_out(1)
        compute(1)
        fire_out(i0 + 1, 1)

        @pl.when(i0 + 3 < NCHUNK)
        def _():
            fire_idx(i0 + 3, 1)
        return carry

    lax.fori_loop(0, (NCHUNK - 1) // 2, pair_body, 0, unroll=False)

    wait_gathers(0)
    wait_out(0)
    compute(0)
    fire_out(NCHUNK - 1, 0)
    wait_out(0)
    wait_out(1)


@jax.jit
def _dist_mult(packed, table, rel):
    mesh = plsc.VectorSubcoreMesh(core_axis_name="c", subcore_axis_name="s")
    return pl.kernel(
        _dist_mult_body,
        out_type=jax.ShapeDtypeStruct((E,), jnp.float32),
        mesh=mesh,
        scratch_types=[
            pltpu.VMEM((NB,), jnp.int32),
            pltpu.VMEM((NB,), jnp.int32),
            pltpu.VMEM((3 * B, WP), jnp.int32),
            pltpu.VMEM((3 * B, WP), jnp.int32),
            pltpu.VMEM((64,), jnp.float32),
            pltpu.VMEM((B,), jnp.float32),
            pltpu.VMEM((B,), jnp.float32),
            pltpu.SemaphoreType.DMA,
            pltpu.SemaphoreType.DMA,
            pltpu.SemaphoreType.DMA,
            pltpu.SemaphoreType.DMA,
            pltpu.SemaphoreType.DMA,
            pltpu.SemaphoreType.DMA,
        ],
        compiler_params=pltpu.CompilerParams(
            needs_layout_passes=False, use_tc_tiling_on_sc=False),
    )(packed, table, rel)


def _pack_bf16(x):
    xb = jnp.pad(x.astype(jnp.bfloat16), ((0, 0), (0, 2 * WP - C)))
    return jax.lax.bitcast_convert_type(
        xb.reshape(x.shape[0], WP, 2), jnp.int32)


def kernel(edge_index, edge_type, initializations, rel_emb):
    table = _pack_bf16(initializations)
    rel = _pack_bf16(rel_emb)
    packed = jnp.concatenate(
        [edge_index[0].reshape(NW * NCHUNK, B),
         edge_index[1].reshape(NW * NCHUNK, B),
         edge_type.reshape(NW * NCHUNK, B)], axis=1)
    return _dist_mult(packed, table, rel)


# speedup vs baseline: 1.2957x; 1.2955x over previous
import jax
import jax.numpy as jnp
from jax import lax
from jax.experimental import pallas as pl
from jax.experimental.pallas import tpu as pltpu
from jax.experimental.pallas import tpu_sc as plsc

N_ENTITIES = 100000
N_RELATIONS = 237
C = 50
CP = 56
E = 1600000
NW = 32
EPW = E // NW
B = 400
NB = 3 * B
NCHUNK = EPW // B
NGROUP = B // 16
CU = 10


def _dist_mult_body(packed_hbm, table_hbm, rel_hbm, out_hbm,
                    idx_v0, idx_v1, rows_v0, rows_v1, rel_v, strip_v,
                    out_v0, out_v1,
                    sem_i0, sem_i1, sem_g0, sem_g1, sem_o0, sem_o1):
    idx_v = (idx_v0, idx_v1)
    rows_v = (rows_v0, rows_v1)
    out_v = (out_v0, out_v1)
    sem_i = (sem_i0, sem_i1)
    sem_g = (sem_g0, sem_g1)
    sem_o = (sem_o0, sem_o1)

    nc = 2
    wid = lax.axis_index("s") * nc + lax.axis_index("c")

    pltpu.sync_copy(rel_hbm, rel_v)

    iota16 = lax.iota(jnp.int32, 16)

    def fire_idx(i, p):
        pltpu.async_copy(packed_hbm.at[wid * NCHUNK + i], idx_v[p], sem_i[p])

    def wait_idx(p):
        pltpu.make_async_copy(packed_hbm.at[0], idx_v[p], sem_i[p]).wait()

    def fire_gathers(p):
        pltpu.async_copy(table_hbm.at[idx_v[p].at[pl.ds(0, B)]],
                         rows_v[p].at[pl.ds(0, B)], sem_g[p])
        pltpu.async_copy(table_hbm.at[idx_v[p].at[pl.ds(B, B)]],
                         rows_v[p].at[pl.ds(B, B)], sem_g[p])

    def wait_gathers(p):
        for j in range(2):
            pltpu.make_async_copy(table_hbm.at[idx_v[p].at[pl.ds(j * B, B)]],
                                  rows_v[p].at[pl.ds(j * B, B)],
                                  sem_g[p]).wait()

    def fire_out(i, p):
        base = wid * EPW + i * B
        pltpu.async_copy(out_v[p], out_hbm.at[pl.ds(base, B)], sem_o[p])

    def wait_out(p):
        pltpu.make_async_copy(out_v[p], out_hbm.at[pl.ds(0, B)],
                              sem_o[p]).wait()

    def compute(p):
        rows = rows_v[p]
        idx = idx_v[p]
        outb = out_v[p]

        zf = jnp.zeros((16,), jnp.float32)
        le16 = jnp.right_shift(iota16, 2)
        lc16 = jnp.bitwise_and(iota16, 3)

        def group_body(g, carry):
            for j in range(4):
                e4 = g * 16 + j * 4 + le16
                be4 = B + e4
                t4 = plsc.load_gather(idx, [2 * B + e4])
                a0 = zf
                a1 = zf
                cvec = lc16
                for k in range(14):
                    s = plsc.load_gather(rows, [e4, cvec])
                    d = plsc.load_gather(rows, [be4, cvec])
                    r = plsc.load_gather(rel_v, [t4, cvec])
                    if k % 2 == 0:
                        a0 = a0 + s * r * d
                    else:
                        a1 = a1 + s * r * d
                    cvec = cvec + 4
                strip_v[pl.ds(j * 16, 16)] = a0 + a1
            a4 = iota16 * 4
            sc = zf
            for lc in range(4):
                sc = sc + plsc.load_gather(strip_v, [a4 + lc])
            outb[pl.ds(g * 16, 16)] = sc
            return carry

        lax.fori_loop(0, NGROUP, group_body, 0, unroll=False)

    fire_idx(0, 0)
    wait_idx(0)
    fire_gathers(0)
    fire_idx(1, 1)

    def pair_body(k, carry):
        i0 = 2 * k
        wait_gathers(0)
        wait_idx(1)
        fire_gathers(1)

        @pl.when(k > 0)
        def _():
            wait_out(0)
        compute(0)
        fire_out(i0, 0)
        fire_idx(i0 + 2, 0)

        wait_gathers(1)
        wait_idx(0)
        fire_gathers(0)

        @pl.when(k > 0)
        def _():
            wait_out(1)
        compute(1)
        fire_out(i0 + 1, 1)

        @pl.when(i0 + 3 < NCHUNK)
        def _():
            fire_idx(i0 + 3, 1)
        return carry

    lax.fori_loop(0, (NCHUNK - 1) // 2, pair_body, 0, unroll=False)

    wait_gathers(0)
    wait_out(0)
    compute(0)
    fire_out(NCHUNK - 1, 0)
    wait_out(0)
    wait_out(1)


@jax.jit
def _dist_mult(packed, table, rel):
    mesh = plsc.VectorSubcoreMesh(core_axis_name="c", subcore_axis_name="s")
    return pl.kernel(
        _dist_mult_body,
        out_type=jax.ShapeDtypeStruct((E,), jnp.float32),
        mesh=mesh,
        scratch_types=[
            pltpu.VMEM((NB,), jnp.int32),
            pltpu.VMEM((NB,), jnp.int32),
            pltpu.VMEM((2 * B, CP), jnp.float32),
            pltpu.VMEM((2 * B, CP), jnp.float32),
            pltpu.VMEM((N_RELATIONS, CP), jnp.float32),
            pltpu.VMEM((64,), jnp.float32),
            pltpu.VMEM((B,), jnp.float32),
            pltpu.VMEM((B,), jnp.float32),
            pltpu.SemaphoreType.DMA,
            pltpu.SemaphoreType.DMA,
            pltpu.SemaphoreType.DMA,
            pltpu.SemaphoreType.DMA,
            pltpu.SemaphoreType.DMA,
            pltpu.SemaphoreType.DMA,
        ],
        compiler_params=pltpu.CompilerParams(
            needs_layout_passes=False, use_tc_tiling_on_sc=False),
    )(packed, table, rel)


def kernel(edge_index, edge_type, initializations, rel_emb):
    table = jnp.pad(initializations, ((0, 0), (0, CP - C)))
    rel = jnp.pad(rel_emb, ((0, 0), (0, CP - C)))
    packed = jnp.concatenate(
        [edge_index[0].reshape(NW * NCHUNK, B),
         edge_index[1].reshape(NW * NCHUNK, B),
         edge_type.reshape(NW * NCHUNK, B)], axis=1)
    return _dist_mult(packed, table, rel)
